# baseline (speedup 1.0000x reference)
import jax
import jax.numpy as jnp
from jax import lax
from jax.experimental import pallas as pl
from jax.experimental.pallas import tpu as pltpu

B = 8
C = 768
H = 32
W = 32
HW = H * W
HALF = C // 2
CB = 96
STEPS = C // CB


def _fill_kernel(row_ref, col_ref, out_ref, pos_ref):
    @pl.when(pl.program_id(0) == 0)
    def _():
        j = lax.broadcasted_iota(jnp.int32, (W, HW), 1)
        lane = lax.broadcasted_iota(jnp.int32, (W, HW), 0)
        sel_w = (j % W == lane).astype(jnp.float32)
        sel_h = (j // W == lane).astype(jnp.float32)
        contract = (((0,), (0,)), ((), ()))
        pos_ref[0:HALF] = lax.dot_general(
            col_ref[0:W, :], sel_w, contract,
            preferred_element_type=jnp.float32).reshape(HALF, 8, 128)
        pos_ref[HALF:C] = lax.dot_general(
            row_ref[0:H, :], sel_h, contract,
            preferred_element_type=jnp.float32).reshape(HALF, 8, 128)

    tile = pos_ref[pl.ds(pl.program_id(0) * CB, CB)]
    for b in range(B):
        out_ref[b] = tile


@jax.jit
def _pos_embed(row_emb, col_emb):
    return pl.pallas_call(
        _fill_kernel,
        grid=(STEPS,),
        in_specs=[
            pl.BlockSpec(row_emb.shape, lambda i: (0, 0)),
            pl.BlockSpec(col_emb.shape, lambda i: (0, 0)),
        ],
        out_specs=pl.BlockSpec((B, CB, 8, 128), lambda i: (0, i, 0, 0)),
        out_shape=jax.ShapeDtypeStruct((B, C, 8, 128), jnp.float32),
        scratch_shapes=[pltpu.VMEM((C, 8, 128), jnp.float32)],
    )(row_emb, col_emb)


def kernel(x, row_emb, col_emb):
    out = _pos_embed(row_emb, col_emb)
    return out.reshape(B, C, H, W)


# speedup vs baseline: 2.0051x; 2.0051x over previous
import jax
import jax.numpy as jnp
from jax.experimental import pallas as pl

B = 8
C = 768
H = 32
W = 32
HALF = C // 2


def _fill_kernel(row_ref, col_ref, out_ref):
    h = pl.program_id(0)
    col = col_ref[0:W, :]
    out_ref[:, 0, :, 0:HALF] = jnp.broadcast_to(col[None], (B, W, HALF))
    row = row_ref[pl.ds(h, 1), :]
    out_ref[:, 0, :, HALF:C] = jnp.broadcast_to(row[None], (B, W, HALF))


@jax.jit
def _pos_embed(row_emb, col_emb):
    return pl.pallas_call(
        _fill_kernel,
        grid=(H,),
        in_specs=[
            pl.BlockSpec(row_emb.shape, lambda i: (0, 0)),
            pl.BlockSpec(col_emb.shape, lambda i: (0, 0)),
        ],
        out_specs=pl.BlockSpec((B, 1, W, C), lambda i: (0, i, 0, 0)),
        out_shape=jax.ShapeDtypeStruct((B, H, W, C), jnp.float32),
    )(row_emb, col_emb)


def kernel(x, row_emb, col_emb):
    out = _pos_embed(row_emb, col_emb)
    return jnp.transpose(out, (0, 3, 1, 2))


# speedup vs baseline: 3.6139x; 1.8023x over previous
import jax
import jax.numpy as jnp
from jax.experimental import pallas as pl

B = 8
C = 768
H = 32
W = 32
HALF = C // 2


def _fill_kernel(row_ref, col_ref, out_ref):
    col = col_ref[0:W, :]
    out_ref[0, :, :, 0:HALF] = jnp.broadcast_to(col[None], (H, W, HALF))
    rows = row_ref[0:H, :]
    out_ref[0, :, :, HALF:C] = jnp.broadcast_to(
        rows[:, None, :], (H, W, HALF))


@jax.jit
def _pos_embed(row_emb, col_emb):
    return pl.pallas_call(
        _fill_kernel,
        grid=(B,),
        in_specs=[
            pl.BlockSpec(row_emb.shape, lambda i: (0, 0)),
            pl.BlockSpec(col_emb.shape, lambda i: (0, 0)),
        ],
        out_specs=pl.BlockSpec((1, H, W, C), lambda i: (i, 0, 0, 0)),
        out_shape=jax.ShapeDtypeStruct((B, H, W, C), jnp.float32),
    )(row_emb, col_emb)


def kernel(x, row_emb, col_emb):
    out = _pos_embed(row_emb, col_emb)
    return jnp.transpose(out, (0, 3, 1, 2))
